# baseline (speedup 1.0000x reference)
import functools

import jax
import jax.numpy as jnp
from jax import lax
from jax.experimental import pallas as pl
from jax.experimental.pallas import tpu as pltpu
from jax.experimental.pallas import tpu_sc as plsc

L = 8
K = 1024
D = 128
LANES = 16

NC = 2
NS = 16
NW = NC * NS

CT = 16
GBUF = 4
OBUF = 2


def _build(n_tokens, n_utt):
    tpw = n_tokens // NW
    n_chunks = tpw // CT
    wpu = NW // n_utt
    mesh = plsc.VectorSubcoreMesh(core_axis_name="c", subcore_axis_name="s")

    @functools.partial(
        pl.kernel,
        mesh=mesh,
        compiler_params=pltpu.CompilerParams(
            use_tc_tiling_on_sc=False, needs_layout_passes=False),
        out_type=jax.ShapeDtypeStruct((n_tokens, D), jnp.float32),
        scratch_types=[
            pltpu.VMEM((L, tpw), jnp.int32),
            pltpu.VMEM((n_chunks, CT * L), jnp.int32),
            pltpu.VMEM((GBUF, CT * L, D // 2), jnp.int32),
            pltpu.VMEM((OBUF, CT, D), jnp.float32),
            pltpu.SemaphoreType.DMA,
            pltpu.SemaphoreType.DMA,
            pltpu.SemaphoreType.DMA,
            pltpu.SemaphoreType.DMA,
            pltpu.SemaphoreType.DMA,
            pltpu.SemaphoreType.DMA,
        ],
    )
    def lookup(idx_hbm, table_hbm, out_hbm, idx_lv, idx_v, rows_v, out_v,
               g0, g1, g2, g3, o0, o1):
        gsems = (g0, g1, g2, g3)
        osems = (o0, o1)
        wid = lax.axis_index("s") * NC + lax.axis_index("c")
        base = wid * tpw
        u = wid // wpu
        wtok0 = (wid % wpu) * tpw
        ii = lax.iota(jnp.int32, LANES) * L

        pltpu.sync_copy(idx_hbm.at[u, :, pl.ds(wtok0, tpw)], idx_lv)

        def interleave(ci, _):
            row = idx_v.at[ci]
            for l in range(L):
                v = idx_lv[l, pl.ds(ci * CT, CT)]
                plsc.store_scatter(row, [ii + l], v + l * K)
            return 0
        lax.fori_loop(0, n_chunks, interleave, 0)

        def gather(ci, b):
            pltpu.async_copy(
                table_hbm.at[idx_v.at[ci]], rows_v.at[b], gsems[b])

        def gather_wait(ci, b):
            pltpu.make_async_copy(
                table_hbm.at[idx_v.at[ci]], rows_v.at[b], gsems[b]).wait()

        def out_start(ci, b):
            pltpu.async_copy(
                out_v.at[b], out_hbm.at[pl.ds(base + ci * CT, CT)], osems[b])

        def out_wait(ci, b):
            pltpu.make_async_copy(
                out_v.at[b], out_hbm.at[pl.ds(base + ci * CT, CT)],
                osems[b]).wait()

        def process(ci, b, ob):
            gather_wait(ci, b)

            @pl.when(ci >= OBUF)
            def _():
                out_wait(ci - OBUF, ob)

            hi_mask = jnp.int32(-65536)

            def decode(xi):
                lo = lax.bitcast_convert_type(
                    lax.shift_left(xi, 16), jnp.float32)
                hi = lax.bitcast_convert_type(
                    lax.bitwise_and(xi, hi_mask), jnp.float32)
                return lo, hi

            @plsc.parallel_loop(0, CT, 1, unroll=4)
            def _acc(t):
                orow = out_v.at[ob, t]
                for j in range(D // (2 * LANES)):
                    sl = pl.ds(j * LANES, LANES)
                    sa, sb = decode(rows_v[b, t * L, sl])
                    for l in range(1, L):
                        a, c = decode(rows_v[b, t * L + l, sl])
                        sa = sa + a
                        sb = sb + c
                    orow[pl.ds(j * LANES, LANES)] = sa
                    orow[pl.ds(D // 2 + j * LANES, LANES)] = sb

            out_start(ci, ob)

        gather(0, 0)
        gather(1, 1)

        def chunk_group(g, _):
            ci = g * GBUF
            for b in range(GBUF):

                @pl.when(ci + b + 2 < n_chunks)
                def _():
                    gather(ci + b + 2, (b + 2) % GBUF)

                process(ci + b, b, b % OBUF)
            return 0
        lax.fori_loop(0, n_chunks // GBUF, chunk_group, 0)

        for b in range(OBUF):
            out_wait(n_chunks - OBUF + b, b)

    return lookup


def kernel(x_list, weight):
    b, t, l = x_list.shape
    n = b * t
    xt = x_list.transpose(0, 2, 1)
    wb = jax.lax.bitcast_convert_type(weight, jnp.int32)
    a, b2 = wb[:, :, : D // 2], wb[:, :, D // 2:]
    rnd_a = jax.lax.shift_right_logical(
        a + 0x7FFF + jax.lax.bitwise_and(jax.lax.shift_right_logical(a, 16), 1),
        16)
    rnd_b = jax.lax.bitwise_and(
        b2 + 0x7FFF
        + jax.lax.bitwise_and(jax.lax.shift_right_logical(b2, 16), 1),
        jnp.int32(-65536))
    table = jax.lax.bitwise_or(rnd_a, rnd_b).reshape(l * K, D // 2)
    out = _build(n, b)(xt, table)
    return out.reshape(b, t, D)


# speedup vs baseline: 1.0390x; 1.0390x over previous
import functools

import jax
import jax.numpy as jnp
from jax import lax
from jax.experimental import pallas as pl
from jax.experimental.pallas import tpu as pltpu
from jax.experimental.pallas import tpu_sc as plsc

L = 8
K = 1024
D = 128
LANES = 16

NC = 2
NS = 16
NW = NC * NS

CT = 16
GBUF = 4
OBUF = 2


def _build(n_tokens, n_utt):
    tpw = n_tokens // NW
    n_chunks = tpw // CT
    wpu = NW // n_utt
    mesh = plsc.VectorSubcoreMesh(core_axis_name="c", subcore_axis_name="s")

    @functools.partial(
        pl.kernel,
        mesh=mesh,
        compiler_params=pltpu.CompilerParams(
            use_tc_tiling_on_sc=False, needs_layout_passes=False),
        out_type=jax.ShapeDtypeStruct((n_tokens, D), jnp.float32),
        scratch_types=[
            pltpu.VMEM((L, tpw), jnp.int32),
            pltpu.VMEM((n_chunks, CT * L), jnp.int32),
            pltpu.VMEM((GBUF, CT * L, D // 2), jnp.int32),
            pltpu.VMEM((OBUF, CT, D), jnp.float32),
            pltpu.SemaphoreType.DMA,
            pltpu.SemaphoreType.DMA,
            pltpu.SemaphoreType.DMA,
            pltpu.SemaphoreType.DMA,
            pltpu.SemaphoreType.DMA,
            pltpu.SemaphoreType.DMA,
        ],
    )
    def lookup(idx_hbm, table_hbm, out_hbm, idx_lv, idx_v, rows_v, out_v,
               g0, g1, g2, g3, o0, o1):
        gsems = (g0, g1, g2, g3)
        osems = (o0, o1)
        wid = lax.axis_index("s") * NC + lax.axis_index("c")
        base = wid * tpw
        u = wid // wpu
        wtok0 = (wid % wpu) * tpw
        ii = lax.iota(jnp.int32, LANES) * L

        pltpu.sync_copy(idx_hbm.at[u, :, pl.ds(wtok0, tpw)], idx_lv)

        def interleave(ci, _):
            row = idx_v.at[ci]
            for l in range(L):
                v = idx_lv[l, pl.ds(ci * CT, CT)]
                plsc.store_scatter(row, [ii + l], v + l * K)
            return 0

        def gather(ci, b):
            pltpu.async_copy(
                table_hbm.at[idx_v.at[ci]], rows_v.at[b], gsems[b])

        def gather_wait(ci, b):
            pltpu.make_async_copy(
                table_hbm.at[idx_v.at[ci]], rows_v.at[b], gsems[b]).wait()

        def out_start(ci, b):
            pltpu.async_copy(
                out_v.at[b], out_hbm.at[pl.ds(base + ci * CT, CT)], osems[b])

        def out_wait(ci, b):
            pltpu.make_async_copy(
                out_v.at[b], out_hbm.at[pl.ds(base + ci * CT, CT)],
                osems[b]).wait()

        def process(ci, b, ob):
            gather_wait(ci, b)

            @pl.when(ci >= OBUF)
            def _():
                out_wait(ci - OBUF, ob)

            hi_mask = jnp.int32(-65536)

            def decode(xi):
                lo = lax.bitcast_convert_type(
                    lax.shift_left(xi, 16), jnp.float32)
                hi = lax.bitcast_convert_type(
                    lax.bitwise_and(xi, hi_mask), jnp.float32)
                return lo, hi

            @plsc.parallel_loop(0, CT, 1, unroll=2)
            def _acc(t):
                orow = out_v.at[ob, t]
                for j in range(D // (2 * LANES)):
                    sl = pl.ds(j * LANES, LANES)
                    sa, sb = decode(rows_v[b, t * L, sl])
                    for l in range(1, L):
                        a, c = decode(rows_v[b, t * L + l, sl])
                        sa = sa + a
                        sb = sb + c
                    orow[pl.ds(j * LANES, LANES)] = sa
                    orow[pl.ds(D // 2 + j * LANES, LANES)] = sb

            out_start(ci, ob)

        interleave(0, 0)
        gather(0, 0)
        interleave(1, 0)
        gather(1, 1)
        lax.fori_loop(2, n_chunks, interleave, 0)

        def chunk_group(g, _):
            ci = g * GBUF
            for b in range(GBUF):

                @pl.when(ci + b + 2 < n_chunks)
                def _():
                    gather(ci + b + 2, (b + 2) % GBUF)

                process(ci + b, b, b % OBUF)
            return 0
        lax.fori_loop(0, n_chunks // GBUF, chunk_group, 0)

        for b in range(OBUF):
            out_wait(n_chunks - OBUF + b, b)

    return lookup


def kernel(x_list, weight):
    b, t, l = x_list.shape
    n = b * t
    xt = x_list.transpose(0, 2, 1)
    wb = jax.lax.bitcast_convert_type(weight, jnp.int32)
    a, b2 = wb[:, :, : D // 2], wb[:, :, D // 2:]
    rnd_a = jax.lax.shift_right_logical(
        a + 0x7FFF + jax.lax.bitwise_and(jax.lax.shift_right_logical(a, 16), 1),
        16)
    rnd_b = jax.lax.bitwise_and(
        b2 + 0x7FFF
        + jax.lax.bitwise_and(jax.lax.shift_right_logical(b2, 16), 1),
        jnp.int32(-65536))
    table = jax.lax.bitwise_or(rnd_a, rnd_b).reshape(l * K, D // 2)
    out = _build(n, b)(xt, table)
    return out.reshape(b, t, D)


# speedup vs baseline: 1.0393x; 1.0003x over previous
import functools

import jax
import jax.numpy as jnp
from jax import lax
from jax.experimental import pallas as pl
from jax.experimental.pallas import tpu as pltpu
from jax.experimental.pallas import tpu_sc as plsc

L = 8
K = 1024
D = 128
LANES = 16

NC = 2
NS = 16
NW = NC * NS

CT = 16
GBUF = 4
OBUF = 4


def _build(n_tokens, n_utt):
    tpw = n_tokens // NW
    n_chunks = tpw // CT
    wpu = NW // n_utt
    mesh = plsc.VectorSubcoreMesh(core_axis_name="c", subcore_axis_name="s")

    @functools.partial(
        pl.kernel,
        mesh=mesh,
        compiler_params=pltpu.CompilerParams(
            use_tc_tiling_on_sc=False, needs_layout_passes=False),
        out_type=jax.ShapeDtypeStruct((n_tokens, D), jnp.float32),
        scratch_types=[
            pltpu.VMEM((L, tpw), jnp.int32),
            pltpu.VMEM((n_chunks, CT * L), jnp.int32),
            pltpu.VMEM((GBUF, CT * L, D // 2), jnp.int32),
            pltpu.VMEM((OBUF, CT, D), jnp.float32),
            pltpu.SemaphoreType.DMA,
            pltpu.SemaphoreType.DMA,
            pltpu.SemaphoreType.DMA,
            pltpu.SemaphoreType.DMA,
            pltpu.SemaphoreType.DMA,
            pltpu.SemaphoreType.DMA,
            pltpu.SemaphoreType.DMA,
            pltpu.SemaphoreType.DMA,
        ],
    )
    def lookup(idx_hbm, table_hbm, out_hbm, idx_lv, idx_v, rows_v, out_v,
               g0, g1, g2, g3, o0, o1, o2, o3):
        gsems = (g0, g1, g2, g3)
        osems = (o0, o1, o2, o3)
        wid = lax.axis_index("s") * NC + lax.axis_index("c")
        base = wid * tpw
        u = wid // wpu
        wtok0 = (wid % wpu) * tpw
        ii = lax.iota(jnp.int32, LANES) * L

        pltpu.sync_copy(idx_hbm.at[u, :, pl.ds(wtok0, tpw)], idx_lv)

        def interleave(ci, _):
            row = idx_v.at[ci]
            for l in range(L):
                v = idx_lv[l, pl.ds(ci * CT, CT)]
                plsc.store_scatter(row, [ii + l], v + l * K)
            return 0

        def gather(ci, b):
            pltpu.async_copy(
                table_hbm.at[idx_v.at[ci]], rows_v.at[b], gsems[b])

        def gather_wait(ci, b):
            pltpu.make_async_copy(
                table_hbm.at[idx_v.at[ci]], rows_v.at[b], gsems[b]).wait()

        def out_start(ci, b):
            pltpu.async_copy(
                out_v.at[b], out_hbm.at[pl.ds(base + ci * CT, CT)], osems[b])

        def out_wait(ci, b):
            pltpu.make_async_copy(
                out_v.at[b], out_hbm.at[pl.ds(base + ci * CT, CT)],
                osems[b]).wait()

        def process(ci, b, ob):
            gather_wait(ci, b)

            @pl.when(ci >= OBUF)
            def _():
                out_wait(ci - OBUF, ob)

            hi_mask = jnp.int32(-65536)

            def decode(xi):
                lo = lax.bitcast_convert_type(
                    lax.shift_left(xi, 16), jnp.float32)
                hi = lax.bitcast_convert_type(
                    lax.bitwise_and(xi, hi_mask), jnp.float32)
                return lo, hi

            @plsc.parallel_loop(0, CT, 1, unroll=2)
            def _acc(t):
                orow = out_v.at[ob, t]
                for j in range(D // (2 * LANES)):
                    sl = pl.ds(j * LANES, LANES)
                    sa, sb = decode(rows_v[b, t * L, sl])
                    for l in range(1, L):
                        a, c = decode(rows_v[b, t * L + l, sl])
                        sa = sa + a
                        sb = sb + c
                    orow[pl.ds(j * LANES, LANES)] = sa
                    orow[pl.ds(D // 2 + j * LANES, LANES)] = sb

            out_start(ci, ob)

        interleave(0, 0)
        gather(0, 0)
        interleave(1, 0)
        gather(1, 1)
        lax.fori_loop(2, n_chunks, interleave, 0)

        def chunk_group(g, _):
            ci = g * GBUF
            for b in range(GBUF):

                @pl.when(ci + b + 2 < n_chunks)
                def _():
                    gather(ci + b + 2, (b + 2) % GBUF)

                process(ci + b, b, b)
            return 0
        lax.fori_loop(0, n_chunks // GBUF, chunk_group, 0)

        for b in range(OBUF):
            out_wait(n_chunks - OBUF + b, b)

    return lookup


def kernel(x_list, weight):
    b, t, l = x_list.shape
    n = b * t
    xt = x_list.transpose(0, 2, 1)
    wb = jax.lax.bitcast_convert_type(weight, jnp.int32)
    a, b2 = wb[:, :, : D // 2], wb[:, :, D // 2:]
    rnd_a = jax.lax.shift_right_logical(
        a + 0x7FFF + jax.lax.bitwise_and(jax.lax.shift_right_logical(a, 16), 1),
        16)
    rnd_b = jax.lax.bitwise_and(
        b2 + 0x7FFF
        + jax.lax.bitwise_and(jax.lax.shift_right_logical(b2, 16), 1),
        jnp.int32(-65536))
    table = jax.lax.bitwise_or(rnd_a, rnd_b).reshape(l * K, D // 2)
    out = _build(n, b)(xt, table)
    return out.reshape(b, t, D)


# speedup vs baseline: 1.0394x; 1.0002x over previous
import functools

import jax
import jax.numpy as jnp
from jax import lax
from jax.experimental import pallas as pl
from jax.experimental.pallas import tpu as pltpu
from jax.experimental.pallas import tpu_sc as plsc

L = 8
K = 1024
D = 128
LANES = 16

NC = 2
NS = 16
NW = NC * NS

CT = 32
GBUF = 4
OBUF = 2


def _build(n_tokens, n_utt):
    tpw = n_tokens // NW
    n_chunks = tpw // CT
    wpu = NW // n_utt
    mesh = plsc.VectorSubcoreMesh(core_axis_name="c", subcore_axis_name="s")

    @functools.partial(
        pl.kernel,
        mesh=mesh,
        compiler_params=pltpu.CompilerParams(
            use_tc_tiling_on_sc=False, needs_layout_passes=False),
        out_type=jax.ShapeDtypeStruct((n_tokens, D), jnp.float32),
        scratch_types=[
            pltpu.VMEM((L, tpw), jnp.int32),
            pltpu.VMEM((tpw * L // 128, 128), jnp.int32),
            pltpu.VMEM((GBUF, CT * L, D // 2), jnp.int32),
            pltpu.VMEM((OBUF, CT, D), jnp.float32),
            pltpu.SemaphoreType.DMA,
            pltpu.SemaphoreType.DMA,
            pltpu.SemaphoreType.DMA,
            pltpu.SemaphoreType.DMA,
            pltpu.SemaphoreType.DMA,
            pltpu.SemaphoreType.DMA,
        ],
    )
    def lookup(idx_hbm, table_hbm, out_hbm, idx_lv, idx_v, rows_v, out_v,
               g0, g1, g2, g3, o0, o1):
        gsems = (g0, g1, g2, g3)
        osems = (o0, o1)
        wid = lax.axis_index("s") * NC + lax.axis_index("c")
        base = wid * tpw
        u = wid // wpu
        wtok0 = (wid % wpu) * tpw
        ii = lax.iota(jnp.int32, LANES) * L

        pltpu.sync_copy(idx_hbm.at[u, :, pl.ds(wtok0, tpw)], idx_lv)

        def interleave(r, _):
            row = idx_v.at[r]
            for l in range(L):
                v = idx_lv[l, pl.ds(r * (128 // L), 128 // L)]
                plsc.store_scatter(row, [ii + l], v + l * K)
            return 0

        def gather(ci, b):
            for h in range(2):
                pltpu.async_copy(
                    table_hbm.at[idx_v.at[2 * ci + h]],
                    rows_v.at[b, pl.ds(128 * h, 128)], gsems[b])

        def gather_wait(ci, b):
            for h in range(2):
                pltpu.make_async_copy(
                    table_hbm.at[idx_v.at[2 * ci + h]],
                    rows_v.at[b, pl.ds(128 * h, 128)], gsems[b]).wait()

        def out_start(ci, b):
            pltpu.async_copy(
                out_v.at[b], out_hbm.at[pl.ds(base + ci * CT, CT)], osems[b])

        def out_wait(ci, b):
            pltpu.make_async_copy(
                out_v.at[b], out_hbm.at[pl.ds(base + ci * CT, CT)],
                osems[b]).wait()

        def process(ci, b, ob):
            gather_wait(ci, b)

            @pl.when(ci >= OBUF)
            def _():
                out_wait(ci - OBUF, ob)

            hi_mask = jnp.int32(-65536)

            def decode(xi):
                lo = lax.bitcast_convert_type(
                    lax.shift_left(xi, 16), jnp.float32)
                hi = lax.bitcast_convert_type(
                    lax.bitwise_and(xi, hi_mask), jnp.float32)
                return lo, hi

            @plsc.parallel_loop(0, CT, 1, unroll=2)
            def _acc(t):
                orow = out_v.at[ob, t]
                for j in range(D // (2 * LANES)):
                    sl = pl.ds(j * LANES, LANES)
                    sa, sb = decode(rows_v[b, t * L, sl])
                    for l in range(1, L):
                        a, c = decode(rows_v[b, t * L + l, sl])
                        sa = sa + a
                        sb = sb + c
                    orow[pl.ds(j * LANES, LANES)] = sa
                    orow[pl.ds(D // 2 + j * LANES, LANES)] = sb

            out_start(ci, ob)

        interleave(0, 0)
        interleave(1, 0)
        gather(0, 0)
        interleave(2, 0)
        interleave(3, 0)
        gather(1, 1)
        lax.fori_loop(4, tpw * L // 128, interleave, 0)

        def chunk_group(g, _):
            ci = g * GBUF
            for b in range(GBUF):

                @pl.when(ci + b + 2 < n_chunks)
                def _():
                    gather(ci + b + 2, (b + 2) % GBUF)

                process(ci + b, b, b % OBUF)
            return 0
        lax.fori_loop(0, n_chunks // GBUF, chunk_group, 0)

        for b in range(OBUF):
            out_wait(n_chunks - OBUF + b, b)

    return lookup


def kernel(x_list, weight):
    b, t, l = x_list.shape
    n = b * t
    xt = x_list.transpose(0, 2, 1)
    wb = jax.lax.bitcast_convert_type(weight, jnp.int32)
    a, b2 = wb[:, :, : D // 2], wb[:, :, D // 2:]
    rnd_a = jax.lax.shift_right_logical(
        a + 0x7FFF + jax.lax.bitwise_and(jax.lax.shift_right_logical(a, 16), 1),
        16)
    rnd_b = jax.lax.bitwise_and(
        b2 + 0x7FFF
        + jax.lax.bitwise_and(jax.lax.shift_right_logical(b2, 16), 1),
        jnp.int32(-65536))
    table = jax.lax.bitwise_or(rnd_a, rnd_b).reshape(l * K, D // 2)
    out = _build(n, b)(xt, table)
    return out.reshape(b, t, D)
